# Initial kernel scaffold; baseline (speedup 1.0000x reference)
import jax
import jax.numpy as jnp
from jax.experimental import pallas as pl


def kernel(label, emb):
    raise NotImplementedError("write your pallas kernel here")


# speedup vs baseline: 4.8937x; 4.8937x over previous
import functools

import jax
import jax.numpy as jnp
from jax import lax
from jax.experimental import pallas as pl
from jax.experimental.pallas import tpu as pltpu
from jax.experimental.pallas import tpu_sc as plsc

NC = 2
NS = 16
NW = NC * NS
L = 16

D = 5
TABLE_ROWS = 1000

N_LABELS = 16384 * 200
PER_W = N_LABELS // NW
CHUNK = 6400
ROUNDS = PER_W // CHUNK
GROUPS = CHUNK // L


def _sc_body(label_h, emb_h, out_h, table_v, lbl_v, out_v):
    wid = lax.axis_index("s") * NC + lax.axis_index("c")
    base = wid * PER_W

    pltpu.sync_copy(emb_h, table_v)

    iota5 = lax.iota(jnp.int32, L) * D

    def round_body(r, carry):
        lbase = base + r * CHUNK
        pltpu.sync_copy(label_h.at[pl.ds(lbase, CHUNK)], lbl_v)

        def group_body(g, c):
            lbl = lbl_v[pl.ds(g * L, L)]
            src = lbl * D
            ob = g * (L * D)
            for k in range(D):
                vals = plsc.load_gather(table_v, [src + k])
                plsc.store_scatter(out_v, [iota5 + (ob + k)], vals)
            return c

        lax.fori_loop(0, GROUPS, group_body, 0)
        pltpu.sync_copy(out_v, out_h.at[pl.ds(lbase * D, CHUNK * D)])
        return carry

    lax.fori_loop(0, ROUNDS, round_body, 0)


def kernel(label, emb):
    n_rows, n_cols = label.shape
    label_flat = label.reshape(-1).astype(jnp.int32)
    emb_flat = emb.reshape(-1)

    mesh = plsc.VectorSubcoreMesh(core_axis_name="c", subcore_axis_name="s")
    run = pl.kernel(
        _sc_body,
        out_type=jax.ShapeDtypeStruct((N_LABELS * D,), jnp.float32),
        mesh=mesh,
        compiler_params=pltpu.CompilerParams(needs_layout_passes=False),
        scratch_types=[
            pltpu.VMEM((TABLE_ROWS * D,), jnp.float32),
            pltpu.VMEM((CHUNK,), jnp.int32),
            pltpu.VMEM((CHUNK * D,), jnp.float32),
        ],
    )
    out_flat = run(label_flat, emb_flat)
    return out_flat.reshape(n_rows, n_cols, D)


# speedup vs baseline: 5.1690x; 1.0563x over previous
import functools

import jax
import jax.numpy as jnp
from jax import lax
from jax.experimental import pallas as pl
from jax.experimental.pallas import tpu as pltpu
from jax.experimental.pallas import tpu_sc as plsc

NC = 2
NS = 16
NW = NC * NS
L = 16

D = 5
TABLE_ROWS = 1000

N_LABELS = 16384 * 200
PER_W = N_LABELS // NW
CHUNK = 6400
ROUNDS = PER_W // CHUNK
GROUPS = CHUNK // L


def _sc_body(label_h, emb_h, out_h, table_v, lbl_v, out_v):
    wid = lax.axis_index("s") * NC + lax.axis_index("c")
    base = wid * PER_W

    pltpu.sync_copy(emb_h, table_v)

    iota5k = [lax.iota(jnp.int32, L) * D + k for k in range(D)]

    def round_body(r, carry):
        lbase = base + r * CHUNK
        pltpu.sync_copy(label_h.at[pl.ds(lbase, CHUNK)], lbl_v)

        @plsc.parallel_loop(0, GROUPS, unroll=8)
        def group_body(g):
            lbl = lbl_v[pl.ds(g * L, L)]
            src = lbl * D
            ob = g * (L * D)
            for k in range(D):
                vals = plsc.load_gather(table_v, [src + k])
                plsc.store_scatter(out_v, [iota5k[k] + ob], vals)

        pltpu.sync_copy(out_v, out_h.at[pl.ds(lbase * D, CHUNK * D)])
        return carry

    lax.fori_loop(0, ROUNDS, round_body, 0)


def kernel(label, emb):
    n_rows, n_cols = label.shape
    label_flat = label.reshape(-1).astype(jnp.int32)
    emb_flat = emb.reshape(-1)

    mesh = plsc.VectorSubcoreMesh(core_axis_name="c", subcore_axis_name="s")
    run = pl.kernel(
        _sc_body,
        out_type=jax.ShapeDtypeStruct((N_LABELS * D,), jnp.float32),
        mesh=mesh,
        compiler_params=pltpu.CompilerParams(needs_layout_passes=False),
        scratch_types=[
            pltpu.VMEM((TABLE_ROWS * D,), jnp.float32),
            pltpu.VMEM((CHUNK,), jnp.int32),
            pltpu.VMEM((CHUNK * D,), jnp.float32),
        ],
    )
    out_flat = run(label_flat, emb_flat)
    return out_flat.reshape(n_rows, n_cols, D)


# speedup vs baseline: 77.0399x; 14.9043x over previous
import jax
import jax.numpy as jnp
from jax import lax
from jax.experimental import pallas as pl
from jax.experimental.pallas import tpu as pltpu
from jax.experimental.pallas import tpu_sc as plsc

NC = 2
NS = 16
NW = NC * NS
L = 16

D = 5
TABLE_ROWS = 1000

N_I = 16384
N_J = 200

W_COLS = N_I // NW
IBLK = 128
NIB = W_COLS // IBLK
R_ROWS = 8
ROUNDS = N_J // R_ROWS
GROUPS = IBLK // L


def _sc_body(label_h, emb_h, out_h, table_v, lbl_v, out_v):
    wid = lax.axis_index("s") * NC + lax.axis_index("c")
    iw = wid * W_COLS

    pltpu.sync_copy(emb_h, table_v)

    iota200 = lax.iota(jnp.int32, L) * N_J

    def iblock_body(ib, carry):
        i0 = iw + ib * IBLK
        pltpu.sync_copy(label_h.at[pl.ds(i0 * N_J, IBLK * N_J)], lbl_v)

        def round_body(r, c2):
            j0 = r * R_ROWS
            for jj in range(R_ROWS):
                j = j0 + jj

                @plsc.parallel_loop(0, GROUPS, unroll=8)
                def group_body(b):
                    ii0 = b * L
                    lbl = plsc.load_gather(lbl_v, [iota200 + (ii0 * N_J + j)])
                    src = lbl * D
                    for k in range(D):
                        vals = plsc.load_gather(table_v, [src + k])
                        out_v[k, jj, pl.ds(ii0, L)] = vals

            for k in range(D):
                pltpu.sync_copy(
                    out_v.at[k],
                    out_h.at[k, pl.ds(j0, R_ROWS), pl.ds(i0, IBLK)],
                )
            return c2

        lax.fori_loop(0, ROUNDS, round_body, 0)
        return carry

    lax.fori_loop(0, NIB, iblock_body, 0)


def kernel(label, emb):
    emb_flat = emb.reshape(-1)
    label_flat = label.reshape(-1).astype(jnp.int32)

    mesh = plsc.VectorSubcoreMesh(core_axis_name="c", subcore_axis_name="s")
    run = pl.kernel(
        _sc_body,
        out_type=jax.ShapeDtypeStruct((D, N_J, N_I), jnp.float32),
        mesh=mesh,
        compiler_params=pltpu.CompilerParams(needs_layout_passes=False),
        scratch_types=[
            pltpu.VMEM((TABLE_ROWS * D,), jnp.float32),
            pltpu.VMEM((IBLK * N_J,), jnp.int32),
            pltpu.VMEM((D, R_ROWS, IBLK), jnp.float32),
        ],
    )
    out_planar = run(label_flat, emb_flat)
    return jnp.transpose(out_planar, (2, 1, 0))


# speedup vs baseline: 127.9994x; 1.6615x over previous
import jax
import jax.numpy as jnp
from jax import lax
from jax.experimental import pallas as pl
from jax.experimental.pallas import tpu as pltpu
from jax.experimental.pallas import tpu_sc as plsc

NC = 2
NS = 16
NW = NC * NS
L = 16

D = 5
TABLE_ROWS = 1000

N_I = 16384
N_J = 200

W_COLS = N_I // NW
IBLK = 256
NIB = W_COLS // IBLK
R_ROWS = 8
ROUNDS = N_J // R_ROWS
GROUPS = IBLK // L


def _sc_body(label_h, emb_h, out_h, table_v, lbl_v, out_v):
    wid = lax.axis_index("s") * NC + lax.axis_index("c")
    iw = wid * W_COLS

    pltpu.sync_copy(emb_h, table_v)

    iota = lax.iota(jnp.int32, L)

    for ib in range(NIB):
        i0 = iw + ib * IBLK
        pltpu.sync_copy(label_h.at[:, pl.ds(i0, IBLK)], lbl_v)

        def round_body(r, carry):
            j0 = r * R_ROWS
            for jj in range(R_ROWS):
                j = j0 + jj

                @plsc.parallel_loop(0, GROUPS, unroll=8)
                def group_body(b):
                    ii0 = b * L
                    lbl = lbl_v[j, pl.ds(ii0, L)]
                    for k in range(D):
                        vals = plsc.load_gather(table_v, [lbl + k * TABLE_ROWS])
                        out_v[k, jj, pl.ds(ii0, L)] = vals

            for k in range(D):
                pltpu.sync_copy(
                    out_v.at[k],
                    out_h.at[k, pl.ds(j0, R_ROWS), pl.ds(i0, IBLK)],
                )
            return carry

        lax.fori_loop(0, ROUNDS, round_body, 0)


def kernel(label, emb):
    table_t = emb.T.reshape(-1)
    label_t = label.T

    mesh = plsc.VectorSubcoreMesh(core_axis_name="c", subcore_axis_name="s")
    run = pl.kernel(
        _sc_body,
        out_type=jax.ShapeDtypeStruct((D, N_J, N_I), jnp.float32),
        mesh=mesh,
        compiler_params=pltpu.CompilerParams(needs_layout_passes=False),
        scratch_types=[
            pltpu.VMEM((TABLE_ROWS * D,), jnp.float32),
            pltpu.VMEM((N_J, IBLK), jnp.int32),
            pltpu.VMEM((D, R_ROWS, IBLK), jnp.float32),
        ],
    )
    out_planar = run(label_t, table_t)
    return jnp.transpose(out_planar, (2, 1, 0))


# speedup vs baseline: 223.6665x; 1.7474x over previous
import jax
import jax.numpy as jnp
from jax import lax
from jax.experimental import pallas as pl
from jax.experimental.pallas import tpu as pltpu
from jax.experimental.pallas import tpu_sc as plsc

NC = 2
NS = 16
NW = NC * NS
L = 16

D = 5
TABLE_ROWS = 1000

N_I = 16384
N_J = 200

W_COLS = N_I // NW
R_ROWS = 8
R_COLS = 128
NTI = W_COLS // R_COLS
NTJ = N_J // R_ROWS
NROUNDS = NTI * NTJ
GROUPS = (R_ROWS * R_COLS) // L


def _sc_body(label_h, emb_h, out_h, table_v, lbl_v, buf_a, buf_b, sem_a, sem_b):
    wid = lax.axis_index("s") * NC + lax.axis_index("c")
    iw = wid * W_COLS

    pltpu.sync_copy(emb_h, table_v)
    pltpu.sync_copy(label_h.at[:, pl.ds(iw, W_COLS)], lbl_v)

    def compute(rho, buf):
        ti = lax.rem(rho, NTI)
        tj = lax.div(rho, NTI)
        j0 = tj * R_ROWS
        i0 = ti * R_COLS

        @plsc.parallel_loop(0, GROUPS, unroll=8)
        def group_body(g):
            jj = g >> 3
            ii0 = (g & 7) * L
            lbl = lbl_v[j0 + jj, pl.ds(i0 + ii0, L)]
            for k in range(D):
                vals = plsc.load_gather(table_v, [lbl + k * TABLE_ROWS])
                buf[k, jj, pl.ds(ii0, L)] = vals

        return j0, i0

    def issue(j0, i0, buf, sem):
        for k in range(D):
            pltpu.async_copy(
                buf.at[k],
                out_h.at[k, pl.ds(j0, R_ROWS), pl.ds(iw + i0, R_COLS)],
                sem,
            )

    def drain(buf, sem):
        for k in range(D):
            pltpu.make_async_copy(
                buf.at[k],
                out_h.at[k, pl.ds(0, R_ROWS), pl.ds(iw, R_COLS)],
                sem,
            ).wait()

    j0, i0 = compute(0, buf_a)
    issue(j0, i0, buf_a, sem_a)
    j0, i0 = compute(1, buf_b)
    issue(j0, i0, buf_b, sem_b)

    def pipe_body(t, carry):
        rho = 2 * t + 2
        drain(buf_a, sem_a)
        j0, i0 = compute(rho, buf_a)
        issue(j0, i0, buf_a, sem_a)
        drain(buf_b, sem_b)
        j0b, i0b = compute(rho + 1, buf_b)
        issue(j0b, i0b, buf_b, sem_b)
        return carry

    lax.fori_loop(0, (NROUNDS - 2) // 2, pipe_body, 0)
    drain(buf_a, sem_a)
    drain(buf_b, sem_b)


def kernel(label, emb):
    table_t = emb.T.reshape(-1)
    label_t = label.T

    mesh = plsc.VectorSubcoreMesh(core_axis_name="c", subcore_axis_name="s")
    run = pl.kernel(
        _sc_body,
        out_type=jax.ShapeDtypeStruct((D, N_J, N_I), jnp.float32),
        mesh=mesh,
        compiler_params=pltpu.CompilerParams(needs_layout_passes=False),
        scratch_types=[
            pltpu.VMEM((TABLE_ROWS * D,), jnp.float32),
            pltpu.VMEM((N_J, W_COLS), jnp.int32),
            pltpu.VMEM((D, R_ROWS, R_COLS), jnp.float32),
            pltpu.VMEM((D, R_ROWS, R_COLS), jnp.float32),
            pltpu.SemaphoreType.DMA,
            pltpu.SemaphoreType.DMA,
        ],
    )
    out_planar = run(label_t, table_t)
    return jnp.transpose(out_planar, (2, 1, 0))
